# baseline (speedup 1.0000x reference)
import jax
import jax.numpy as jnp
from jax import lax
from jax.experimental import pallas as pl
from jax.experimental.pallas import tpu as pltpu

_NB_TF = 4
_N_SEQ_TF = 2
_BLK = 4


def _body(c_ref, x_ref, o_ref):
    i = pl.program_id(0)
    ch, h, w = x_ref.shape[1], x_ref.shape[2], x_ref.shape[3]
    n = ch * h * w
    rows = lax.broadcasted_iota(jnp.int32, (w, w), 0)
    cols = lax.broadcasted_iota(jnp.int32, (w, w), 1)
    antimat = jnp.where(rows + cols == w - 1, 1.0, 0.0).astype(jnp.float32)
    for b in range(_BLK):
        col = i * _BLK + b
        a0 = c_ref[0, col]
        g0 = c_ref[1, col]
        a1 = c_ref[2, col]
        g1 = c_ref[3, col]
        c0 = c_ref[4, col]
        c1 = c_ref[5, col]
        nm0 = c_ref[6, col]
        nm1 = c_ref[7, col]
        fb = c_ref[8, col]
        xb = x_ref[b].reshape(ch * h, w)
        xf = lax.cond(
            fb == 1.0,
            lambda v: jnp.dot(v, antimat, preferred_element_type=jnp.float32),
            lambda v: v, xb)
        m0 = lax.cond(nm0 == 1.0, jnp.sum, lambda v: jnp.float32(0.0), xf) / n
        y = lax.cond(
            c0 == 1.0,
            lambda v: jnp.minimum(jnp.maximum(v * a0 + g0 * m0, 0.0), 1.0),
            lambda v: v, xf)
        m1 = lax.cond(nm1 == 1.0, jnp.sum, lambda v: jnp.float32(0.0), y) / n
        z = lax.cond(
            c1 == 1.0,
            lambda v: jnp.minimum(jnp.maximum(v * a1 + g1 * m1, 0.0), 1.0),
            lambda v: v, y)
        o_ref[b] = z.reshape(ch, h, w)


def kernel(x, prob, mag, temp):
    batch = x.shape[0]
    temp_d = lax.stop_gradient(temp)
    mag_d = lax.stop_gradient(mag)
    distrib = jax.nn.softmax(prob * temp_d, axis=0)
    logits = jnp.log(distrib + 1e-12)
    skey = jax.random.key(42)
    samples = jax.random.categorical(
        skey, jnp.broadcast_to(logits, (batch, _NB_TF)), axis=-1,
        shape=(_N_SEQ_TF, batch)
    ).astype(jnp.int32)
    s0, s1 = samples[0], samples[1]

    f = jnp.float32(0.5) + mag_d / jnp.float32(1.0)
    one = jnp.float32(1.0)
    zero = jnp.float32(0.0)

    def coeffs(s):
        c = s >= 2
        a = jnp.where(c, f, one)
        g = jnp.where(s == 3, one - f, zero)
        return a, g, c.astype(jnp.float32), (s == 3).astype(jnp.float32)

    a0, g0, c0, nm0 = coeffs(s0)
    a1, g1, c1, nm1 = coeffs(s1)
    flip = ((s0 == 1) != (s1 == 1)).astype(jnp.float32)
    ctab = jnp.stack([a0, g0, a1, g1, c0, c1, nm0, nm1, flip], axis=0)

    out = pl.pallas_call(
        _body,
        grid=(batch // _BLK,),
        in_specs=[
            pl.BlockSpec(memory_space=pltpu.SMEM),
            pl.BlockSpec((_BLK,) + x.shape[1:], lambda i: (i, 0, 0, 0)),
        ],
        out_specs=pl.BlockSpec((_BLK,) + x.shape[1:], lambda i: (i, 0, 0, 0)),
        out_shape=jax.ShapeDtypeStruct(x.shape, x.dtype),
    )(ctab, x)
    return out
